# baseline (speedup 1.0000x reference)
import functools

import jax
import jax.numpy as jnp
from jax.experimental import pallas as pl
from jax.experimental.pallas import tpu as pltpu
from jax.experimental.pallas import tpu_sc as plsc

BATCH = 16384
EMBED = 64
WINDOW = 128


def _sc_gather(table2, idx):
    num_windows = BATCH // WINDOW
    mesh = plsc.VectorSubcoreMesh(core_axis_name="core",
                                  subcore_axis_name="subcore")

    @functools.partial(
        pl.kernel,
        out_type=jax.ShapeDtypeStruct((BATCH, 2 * EMBED), table2.dtype),
        mesh=mesh,
    )
    def gather_kernel(x_hbm, i_hbm, o_hbm):
        def body(i_vmem, o_vmem):
            pltpu.sync_copy(x_hbm.at[i_vmem.at[0]], o_vmem)

        pltpu.emit_pipeline(
            body,
            grid=(num_windows,),
            in_specs=[pl.BlockSpec((1, WINDOW), index_map=lambda i: (0, i))],
            out_specs=[pl.BlockSpec((WINDOW, 2 * EMBED),
                                    index_map=lambda i: (i, 0))],
            core_axis_name=("core", "subcore"),
            dimension_semantics=(pltpu.PARALLEL,),
        )(i_hbm, o_hbm)

    return gather_kernel(table2, idx.reshape(1, BATCH))


TC_BLOCK = 2048


def _tc_loss_body(f_ref, g_ref, lab_ref, o_ref):
    f = f_ref[...]
    g = g_ref[...]
    par = lab_ref[...] % 2
    c = jnp.where(par == 0, g[:, :EMBED], g[:, EMBED:])
    fn = jnp.sqrt(jnp.sum(f * f, axis=1, keepdims=True))
    f1 = f / jnp.maximum(fn, 1e-12)
    cn = jnp.sqrt(jnp.sum(c * c, axis=1, keepdims=True))
    c1 = c / jnp.maximum(cn, 1e-12)
    num = jnp.sum(f1 * c1, axis=1, keepdims=True)
    d1 = jnp.sqrt(jnp.sum(f1 * f1, axis=1, keepdims=True))
    d2 = jnp.sqrt(jnp.sum(c1 * c1, axis=1, keepdims=True))
    cos = num / jnp.maximum(d1 * d2, 1e-8)
    part = jnp.sum(1.0 - cos, axis=0, keepdims=True) / BATCH

    @pl.when(pl.program_id(0) == 0)
    def _():
        o_ref[...] = jnp.zeros_like(o_ref)

    o_ref[...] += part


def _tc_loss(features, gathered, labels_2d):
    return pl.pallas_call(
        _tc_loss_body,
        grid=(BATCH // TC_BLOCK,),
        in_specs=[
            pl.BlockSpec((TC_BLOCK, EMBED), lambda i: (i, 0)),
            pl.BlockSpec((TC_BLOCK, 2 * EMBED), lambda i: (i, 0)),
            pl.BlockSpec((TC_BLOCK, 1), lambda i: (i, 0)),
        ],
        out_specs=pl.BlockSpec((1, 1), lambda i: (0, 0)),
        out_shape=jax.ShapeDtypeStruct((1, 1), jnp.float32),
    )(features, gathered, labels_2d)


def kernel(features, labels, centers):
    labels32 = labels.astype(jnp.int32)
    table2 = centers.reshape(centers.shape[0] // 2, 2 * EMBED)
    gathered = _sc_gather(table2, labels32 // 2)
    loss = _tc_loss(features, gathered, labels32.reshape(BATCH, 1))
    return loss[0, 0]
